# Initial kernel scaffold; baseline (speedup 1.0000x reference)
import functools

import jax
import jax.numpy as jnp
from jax import lax
from jax.experimental import pallas as pl
from jax.experimental.pallas import tpu as pltpu
from jax.experimental.pallas import tpu_sc as plsc

VOCAB = 100004
MAXLEN = 200
EMB = 50
HID = 128
B = 1024

N_ROWS = B * MAXLEN


def _make_sc_gather(n_rows: int, emb: int, chunk: int = 128):
    info = plsc.get_sparse_core_info()
    nc, ns = info.num_cores, info.num_subcores
    nw = nc * ns
    assert n_rows % nw == 0
    rpw = n_rows // nw
    assert rpw % chunk == 0
    n_chunks = rpw // chunk

    mesh = plsc.VectorSubcoreMesh(core_axis_name="c", subcore_axis_name="s")

    @functools.partial(
        pl.kernel,
        out_type=jax.ShapeDtypeStruct((n_rows, emb), jnp.float32),
        mesh=mesh,
        scratch_types=[
            pltpu.VMEM((chunk,), jnp.int32),
            pltpu.VMEM((chunk, emb), jnp.float32),
            pltpu.SemaphoreType.DMA,
        ],
    )
    def sc_gather(table_hbm, idx_hbm, out_hbm, idx_v, rows_v, sem):
        wid = lax.axis_index("s") * nc + lax.axis_index("c")
        base0 = wid * rpw

        def body(i, carry):
            base = base0 + i * chunk
            pltpu.sync_copy(idx_hbm.at[pl.ds(base, chunk)], idx_v)
            pltpu.async_copy(table_hbm.at[idx_v], rows_v, sem).wait()
            pltpu.sync_copy(rows_v, out_hbm.at[pl.ds(base, chunk)])
            return carry

        lax.fori_loop(0, n_chunks, body, 0)

    return sc_gather


_sc_gather = _make_sc_gather(N_ROWS, EMB)


def _tc_body(we_ref, tt_ref, pos_ref, tok_ref, w_ref, b_ref, g_ref, bt_ref,
             out_ref, *, bb):
    we = we_ref[...]
    tt = tt_ref[...].astype(jnp.float32)
    pos = pos_ref[...]
    tok0 = tok_ref[0:1, :]
    tok1 = tok_ref[1:2, :]
    emb = (we + pos[None]
           + tok0[None]
           + tt[..., None] * (tok1 - tok0)[None])
    x = lax.dot_general(
        emb.reshape(bb * MAXLEN, EMB), w_ref[...],
        (((1,), (0,)), ((), ())),
        preferred_element_type=jnp.float32,
    ) + b_ref[...]
    mean = jnp.mean(x, axis=-1, keepdims=True)
    xc = x - mean
    var = jnp.mean(xc * xc, axis=-1, keepdims=True)
    y = xc * lax.rsqrt(var + 1e-5) * g_ref[...] + bt_ref[...]
    out_ref[...] = y.reshape(bb, MAXLEN, HID)


def _make_tc_fuse(bb: int = 16, interpret: bool = False):
    grid = (B // bb,)
    return pl.pallas_call(
        functools.partial(_tc_body, bb=bb),
        grid=grid,
        in_specs=[
            pl.BlockSpec((bb, MAXLEN, EMB), lambda i: (i, 0, 0)),
            pl.BlockSpec((bb, MAXLEN), lambda i: (i, 0)),
            pl.BlockSpec((MAXLEN, EMB), lambda i: (0, 0)),
            pl.BlockSpec((2, EMB), lambda i: (0, 0)),
            pl.BlockSpec((EMB, HID), lambda i: (0, 0)),
            pl.BlockSpec((1, HID), lambda i: (0, 0)),
            pl.BlockSpec((1, HID), lambda i: (0, 0)),
            pl.BlockSpec((1, HID), lambda i: (0, 0)),
        ],
        out_specs=pl.BlockSpec((bb, MAXLEN, HID), lambda i: (i, 0, 0)),
        out_shape=jax.ShapeDtypeStruct((B, MAXLEN, HID), jnp.float32),
        interpret=interpret,
    )


_tc_fuse = _make_tc_fuse()


def kernel(word_id, token_type, word_table, pos_table, tok_table, W, b,
           gamma, beta):
    idx = word_id.reshape(-1).astype(jnp.int32)
    we = _sc_gather(word_table, idx)
    we = we.reshape(B, MAXLEN, EMB)
    return _tc_fuse(we, token_type.astype(jnp.int32), pos_table, tok_table,
                    W, b.reshape(1, HID), gamma.reshape(1, HID),
                    beta.reshape(1, HID))


# speedup vs baseline: 4.0290x; 4.0290x over previous
import functools

import jax
import jax.numpy as jnp
from jax import lax
from jax.experimental import pallas as pl
from jax.experimental.pallas import tpu as pltpu
from jax.experimental.pallas import tpu_sc as plsc

VOCAB = 100004
MAXLEN = 200
EMB = 50
EMBP = 64
HID = 128
B = 1024

N_ROWS = B * MAXLEN


def _make_sc_gather(n_rows: int, emb: int, chunk: int = 128):
    info = plsc.get_sparse_core_info()
    nc, ns = info.num_cores, info.num_subcores
    nw = nc * ns
    assert n_rows % nw == 0
    rpw = n_rows // nw
    assert rpw % chunk == 0
    n_chunks = rpw // chunk

    mesh = plsc.VectorSubcoreMesh(core_axis_name="c", subcore_axis_name="s",
                                  num_cores=nc, num_subcores=ns)

    @functools.partial(
        pl.kernel,
        out_type=jax.ShapeDtypeStruct((n_rows, emb), jnp.float32),
        mesh=mesh,
        scratch_types=[
            pltpu.VMEM((chunk,), jnp.int32),
            pltpu.VMEM((chunk, emb), jnp.float32),
            pltpu.SemaphoreType.DMA,
        ],
        compiler_params=pltpu.CompilerParams(use_tc_tiling_on_sc=False),
    )
    def sc_gather(table_hbm, idx_hbm, out_hbm, idx_v, rows_v, sem):
        wid = lax.axis_index("s") * nc + lax.axis_index("c")
        base0 = wid * rpw

        def body(i, carry):
            base = base0 + i * chunk
            pltpu.sync_copy(idx_hbm.at[pl.ds(base, chunk)], idx_v)
            pltpu.async_copy(table_hbm.at[idx_v], rows_v, sem).wait()
            pltpu.sync_copy(rows_v, out_hbm.at[pl.ds(base, chunk)])
            return carry

        lax.fori_loop(0, n_chunks, body, 0)

    return sc_gather


_sc_gather_cache = {}


def _sc_gather(table, idx):
    if "k" not in _sc_gather_cache:
        _sc_gather_cache["k"] = _make_sc_gather(N_ROWS, EMBP)
    return _sc_gather_cache["k"](table, idx)


def _tc_body(we_ref, tt_ref, pos_ref, tok_ref, w_ref, b_ref, g_ref, bt_ref,
             out_ref, *, bb):
    we = we_ref[...]
    tt = tt_ref[...].astype(jnp.float32)
    pos = pos_ref[...]
    tok0 = tok_ref[0:1, :]
    tok1 = tok_ref[1:2, :]
    emb = (we + pos[None]
           + tok0[None]
           + tt[..., None] * (tok1 - tok0)[None])
    x = lax.dot_general(
        emb.reshape(bb * MAXLEN, EMBP), w_ref[...],
        (((1,), (0,)), ((), ())),
        preferred_element_type=jnp.float32,
    ) + b_ref[...]
    mean = jnp.mean(x, axis=-1, keepdims=True)
    xc = x - mean
    var = jnp.mean(xc * xc, axis=-1, keepdims=True)
    y = xc * lax.rsqrt(var + 1e-5) * g_ref[...] + bt_ref[...]
    out_ref[...] = y.reshape(bb, MAXLEN, HID)


def _make_tc_fuse(bb: int = 16, interpret: bool = False):
    grid = (B // bb,)
    return pl.pallas_call(
        functools.partial(_tc_body, bb=bb),
        grid=grid,
        in_specs=[
            pl.BlockSpec((bb, MAXLEN, EMBP), lambda i: (i, 0, 0)),
            pl.BlockSpec((bb, MAXLEN), lambda i: (i, 0)),
            pl.BlockSpec((MAXLEN, EMBP), lambda i: (0, 0)),
            pl.BlockSpec((2, EMBP), lambda i: (0, 0)),
            pl.BlockSpec((EMBP, HID), lambda i: (0, 0)),
            pl.BlockSpec((1, HID), lambda i: (0, 0)),
            pl.BlockSpec((1, HID), lambda i: (0, 0)),
            pl.BlockSpec((1, HID), lambda i: (0, 0)),
        ],
        out_specs=pl.BlockSpec((bb, MAXLEN, HID), lambda i: (i, 0, 0)),
        out_shape=jax.ShapeDtypeStruct((B, MAXLEN, HID), jnp.float32),
        interpret=interpret,
    )


_tc_fuse = _make_tc_fuse()


def kernel(word_id, token_type, word_table, pos_table, tok_table, W, b,
           gamma, beta):
    pad = EMBP - EMB
    wt = jnp.pad(word_table, ((0, 0), (0, pad)))
    posp = jnp.pad(pos_table, ((0, 0), (0, pad)))
    tokp = jnp.pad(tok_table, ((0, 0), (0, pad)))
    wp = jnp.pad(W, ((0, pad), (0, 0)))
    idx = word_id.reshape(-1).astype(jnp.int32)
    we = _sc_gather(wt, idx)
    we = we.reshape(B, MAXLEN, EMBP)
    return _tc_fuse(we, token_type.astype(jnp.int32), posp, tokp,
                    wp, b.reshape(1, HID), gamma.reshape(1, HID),
                    beta.reshape(1, HID))


# speedup vs baseline: 5.7087x; 1.4169x over previous
import functools

import jax
import jax.numpy as jnp
from jax import lax
from jax.experimental import pallas as pl
from jax.experimental.pallas import tpu as pltpu
from jax.experimental.pallas import tpu_sc as plsc

VOCAB = 100004
MAXLEN = 200
EMB = 50
HID = 128
B = 1024

N_ROWS = B * MAXLEN
PROJ_BLK = 4096
VPAD = 102400


def _proj_body(wt_ref, w_ref, b_ref, out_ref):
    out_ref[...] = lax.dot_general(
        wt_ref[...], w_ref[...],
        (((1,), (0,)), ((), ())),
        preferred_element_type=jnp.float32,
    ) + b_ref[...]


def _make_proj(interpret: bool = False):
    return pl.pallas_call(
        _proj_body,
        grid=(VPAD // PROJ_BLK,),
        in_specs=[
            pl.BlockSpec((PROJ_BLK, EMB), lambda i: (i, 0)),
            pl.BlockSpec((EMB, HID), lambda i: (0, 0)),
            pl.BlockSpec((1, HID), lambda i: (0, 0)),
        ],
        out_specs=pl.BlockSpec((PROJ_BLK, HID), lambda i: (i, 0)),
        out_shape=jax.ShapeDtypeStruct((VPAD, HID), jnp.float32),
        interpret=interpret,
    )


_proj = _make_proj()


def _make_sc_gather(n_rows: int, width: int, chunk: int = 128):
    info = plsc.get_sparse_core_info()
    nc, ns = info.num_cores, info.num_subcores
    nw = nc * ns
    assert n_rows % nw == 0
    rpw = n_rows // nw
    assert rpw % chunk == 0
    n_chunks = rpw // chunk

    mesh = plsc.VectorSubcoreMesh(core_axis_name="c", subcore_axis_name="s",
                                  num_cores=nc, num_subcores=ns)

    @functools.partial(
        pl.kernel,
        out_type=jax.ShapeDtypeStruct((n_rows, width), jnp.float32),
        mesh=mesh,
        scratch_types=[
            pltpu.VMEM((2, chunk), jnp.int32),
            pltpu.VMEM((2, chunk, width), jnp.float32),
            pltpu.SemaphoreType.DMA((2,)),
        ],
        compiler_params=pltpu.CompilerParams(use_tc_tiling_on_sc=False),
    )
    def sc_gather(table_hbm, idx_hbm, out_hbm, idx_v, rows_v, gsem):
        wid = lax.axis_index("s") * nc + lax.axis_index("c")
        base0 = wid * rpw

        def stage_and_start(i, slot):
            pltpu.sync_copy(idx_hbm.at[pl.ds(base0 + i * chunk, chunk)],
                            idx_v.at[slot])
            pltpu.async_copy(table_hbm.at[idx_v.at[slot]], rows_v.at[slot],
                             gsem.at[slot])

        stage_and_start(0, 0)

        def body(i, carry):
            slot = lax.rem(i, 2)
            nslot = lax.rem(i + 1, 2)

            @pl.when(i + 1 < n_chunks)
            def _():
                stage_and_start(i + 1, nslot)

            pltpu.make_async_copy(table_hbm.at[idx_v.at[slot]],
                                  rows_v.at[slot], gsem.at[slot]).wait()
            pltpu.sync_copy(rows_v.at[slot],
                            out_hbm.at[pl.ds(base0 + i * chunk, chunk)])
            return carry

        lax.fori_loop(0, n_chunks, body, 0)

    return sc_gather


_sc_gather_cache = {}


def _sc_gather(table, idx):
    if "k" not in _sc_gather_cache:
        _sc_gather_cache["k"] = _make_sc_gather(N_ROWS, HID)
    return _sc_gather_cache["k"](table, idx)


def _fin_body(g_ref, tt_ref, pos_ref, tok_ref, w_ref, gam_ref, bet_ref,
              out_ref, *, bb):
    posp = lax.dot_general(
        pos_ref[...] + tok_ref[0:1, :], w_ref[...],
        (((1,), (0,)), ((), ())),
        preferred_element_type=jnp.float32,
    )
    tokd = lax.dot_general(
        tok_ref[1:2, :] - tok_ref[0:1, :], w_ref[...],
        (((1,), (0,)), ((), ())),
        preferred_element_type=jnp.float32,
    )
    tt = tt_ref[...].astype(jnp.float32)
    x = (g_ref[...] + posp[None]
         + tt[..., None] * tokd[None])
    mean = jnp.mean(x, axis=-1, keepdims=True)
    xc = x - mean
    var = jnp.mean(xc * xc, axis=-1, keepdims=True)
    out_ref[...] = (xc * lax.rsqrt(var + 1e-5) * gam_ref[...]
                    + bet_ref[...])


def _make_fin(bb: int = 16, interpret: bool = False):
    return pl.pallas_call(
        functools.partial(_fin_body, bb=bb),
        grid=(B // bb,),
        in_specs=[
            pl.BlockSpec((bb, MAXLEN, HID), lambda i: (i, 0, 0)),
            pl.BlockSpec((bb, MAXLEN), lambda i: (i, 0)),
            pl.BlockSpec((MAXLEN, EMB), lambda i: (0, 0)),
            pl.BlockSpec((2, EMB), lambda i: (0, 0)),
            pl.BlockSpec((EMB, HID), lambda i: (0, 0)),
            pl.BlockSpec((1, HID), lambda i: (0, 0)),
            pl.BlockSpec((1, HID), lambda i: (0, 0)),
        ],
        out_specs=pl.BlockSpec((bb, MAXLEN, HID), lambda i: (i, 0, 0)),
        out_shape=jax.ShapeDtypeStruct((B, MAXLEN, HID), jnp.float32),
        interpret=interpret,
    )


_fin = _make_fin()


def kernel(word_id, token_type, word_table, pos_table, tok_table, W, b,
           gamma, beta):
    proj = _proj(word_table, W, b.reshape(1, HID))
    idx = word_id.reshape(-1).astype(jnp.int32)
    g = _sc_gather(proj, idx)
    g = g.reshape(B, MAXLEN, HID)
    return _fin(g, token_type.astype(jnp.int32), pos_table, tok_table, W,
                gamma.reshape(1, HID), beta.reshape(1, HID))


# speedup vs baseline: 7.1334x; 1.2496x over previous
import functools

import jax
import jax.numpy as jnp
from jax import lax
from jax.experimental import pallas as pl
from jax.experimental.pallas import tpu as pltpu
from jax.experimental.pallas import tpu_sc as plsc

VOCAB = 100004
MAXLEN = 200
EMB = 50
HID = 128
B = 1024

N_ROWS = B * MAXLEN
PROJ_BLK = 4096
VPAD = 102400
SLABS = 2
SLAB_B = B // SLABS
SLAB_ROWS = SLAB_B * MAXLEN


def _proj_body(wtt_ref, w_ref, b_ref, out_ref):
    out_ref[...] = lax.dot_general(
        wtt_ref[...], w_ref[...],
        (((0,), (0,)), ((), ())),
        preferred_element_type=jnp.float32,
    ) + b_ref[...]


def _make_proj(interpret: bool = False):
    return pl.pallas_call(
        _proj_body,
        grid=(VPAD // PROJ_BLK,),
        in_specs=[
            pl.BlockSpec((EMB, PROJ_BLK), lambda i: (0, i)),
            pl.BlockSpec((EMB, HID), lambda i: (0, 0)),
            pl.BlockSpec((1, HID), lambda i: (0, 0)),
        ],
        out_specs=pl.BlockSpec((PROJ_BLK, HID), lambda i: (i, 0)),
        out_shape=jax.ShapeDtypeStruct((VPAD, HID), jnp.float32),
        interpret=interpret,
    )


_proj = _make_proj()


def _make_sc_gather(n_rows: int, width: int, chunk: int = 128):
    info = plsc.get_sparse_core_info()
    nc, ns = info.num_cores, info.num_subcores
    nw = nc * ns
    assert n_rows % nw == 0
    rpw = n_rows // nw
    assert rpw % chunk == 0
    n_chunks = rpw // chunk

    mesh = plsc.VectorSubcoreMesh(core_axis_name="c", subcore_axis_name="s",
                                  num_cores=nc, num_subcores=ns)

    @functools.partial(
        pl.kernel,
        out_type=jax.ShapeDtypeStruct((n_rows, width), jnp.float32),
        mesh=mesh,
        scratch_types=[
            pltpu.VMEM((2, chunk), jnp.int32),
            pltpu.VMEM((2, chunk, width), jnp.float32),
            pltpu.SemaphoreType.DMA((2,)),
        ],
        compiler_params=pltpu.CompilerParams(use_tc_tiling_on_sc=False),
    )
    def sc_gather(table_hbm, idx_hbm, out_hbm, idx_v, rows_v, gsem):
        wid = lax.axis_index("s") * nc + lax.axis_index("c")
        base0 = wid * rpw

        def stage_and_start(i, slot):
            pltpu.sync_copy(idx_hbm.at[pl.ds(base0 + i * chunk, chunk)],
                            idx_v.at[slot])
            pltpu.async_copy(table_hbm.at[idx_v.at[slot]], rows_v.at[slot],
                             gsem.at[slot])

        stage_and_start(0, 0)

        def body(i, carry):
            slot = lax.rem(i, 2)
            nslot = lax.rem(i + 1, 2)

            @pl.when(i + 1 < n_chunks)
            def _():
                stage_and_start(i + 1, nslot)

            pltpu.make_async_copy(table_hbm.at[idx_v.at[slot]],
                                  rows_v.at[slot], gsem.at[slot]).wait()
            pltpu.sync_copy(rows_v.at[slot],
                            out_hbm.at[pl.ds(base0 + i * chunk, chunk)])
            return carry

        lax.fori_loop(0, n_chunks, body, 0)

    return sc_gather


_sc_gather_cache = {}


def _sc_gather(table, idx):
    if "k" not in _sc_gather_cache:
        _sc_gather_cache["k"] = _make_sc_gather(SLAB_ROWS, HID)
    return _sc_gather_cache["k"](table, idx)


def _fin_body(g_ref, tt_ref, pos_ref, tok_ref, w_ref, gam_ref, bet_ref,
              out_ref, *, bb):
    _fin_compute(g_ref, tt_ref, pos_ref, tok_ref, w_ref, gam_ref, bet_ref,
                 out_ref, bb)


def _fin_body_alias(g_ref, tt_ref, pos_ref, tok_ref, w_ref, gam_ref, bet_ref,
                    prev_ref, out_ref, *, bb):
    del prev_ref
    _fin_compute(g_ref, tt_ref, pos_ref, tok_ref, w_ref, gam_ref, bet_ref,
                 out_ref, bb)


def _fin_compute(g_ref, tt_ref, pos_ref, tok_ref, w_ref, gam_ref, bet_ref,
                 out_ref, bb):
    posp = lax.dot_general(
        pos_ref[...] + tok_ref[0:1, :], w_ref[...],
        (((1,), (0,)), ((), ())),
        preferred_element_type=jnp.float32,
    )
    tokd = lax.dot_general(
        tok_ref[1:2, :] - tok_ref[0:1, :], w_ref[...],
        (((1,), (0,)), ((), ())),
        preferred_element_type=jnp.float32,
    )
    tt = tt_ref[...].astype(jnp.float32)
    x = (g_ref[...] + posp[None]
         + tt[..., None] * tokd[None])
    mean = jnp.mean(x, axis=-1, keepdims=True)
    xc = x - mean
    var = jnp.mean(xc * xc, axis=-1, keepdims=True)
    out_ref[...] = (xc * lax.rsqrt(var + 1e-5) * gam_ref[...]
                    + bet_ref[...])


def _make_fin(slab: int, bb: int = 16, interpret: bool = False):
    off = slab * (SLAB_B // bb)
    in_specs = [
        pl.BlockSpec((bb, MAXLEN, HID), lambda i: (i, 0, 0)),
        pl.BlockSpec((bb, MAXLEN), lambda i, _o=off: (i + _o, 0)),
        pl.BlockSpec((MAXLEN, EMB), lambda i: (0, 0)),
        pl.BlockSpec((2, EMB), lambda i: (0, 0)),
        pl.BlockSpec((EMB, HID), lambda i: (0, 0)),
        pl.BlockSpec((1, HID), lambda i: (0, 0)),
        pl.BlockSpec((1, HID), lambda i: (0, 0)),
    ]
    if slab == 0:
        body = functools.partial(_fin_body, bb=bb)
        aliases = {}
    else:
        body = functools.partial(_fin_body_alias, bb=bb)
        in_specs = in_specs + [pl.BlockSpec(memory_space=pl.ANY)]
        aliases = {7: 0}
    return pl.pallas_call(
        body,
        grid=(SLAB_B // bb,),
        in_specs=in_specs,
        out_specs=pl.BlockSpec((bb, MAXLEN, HID),
                               lambda i, _o=off: (i + _o, 0, 0)),
        out_shape=jax.ShapeDtypeStruct((B, MAXLEN, HID), jnp.float32),
        input_output_aliases=aliases,
        interpret=interpret,
    )


_fins = [_make_fin(s) for s in range(SLABS)]


def kernel(word_id, token_type, word_table, pos_table, tok_table, W, b,
           gamma, beta):
    proj = _proj(word_table.T, W, b.reshape(1, HID))
    idx = word_id.reshape(-1).astype(jnp.int32)
    tt = token_type.astype(jnp.int32)
    gs = [_sc_gather(proj, idx[s * SLAB_ROWS:(s + 1) * SLAB_ROWS])
          .reshape(SLAB_B, MAXLEN, HID) for s in range(SLABS)]
    gamma2 = gamma.reshape(1, HID)
    beta2 = beta.reshape(1, HID)
    y = _fins[0](gs[0], tt, pos_table, tok_table, W, gamma2, beta2)
    for s in range(1, SLABS):
        y = _fins[s](gs[s], tt, pos_table, tok_table, W, gamma2, beta2, y)
    return y
